# Initial kernel scaffold; baseline (speedup 1.0000x reference)
import jax
import jax.numpy as jnp
from jax.experimental import pallas as pl


def kernel(input_tensor, mem_keys, mem_values, W1, b1, ln_g, ln_b, W2, b2):
    raise NotImplementedError("write your pallas kernel here")


# speedup vs baseline: 13.7641x; 13.7641x over previous
import functools

import jax
import jax.numpy as jnp
from jax.experimental import pallas as pl

TOPK = 32
T = 1024
D = 128
M = 65536
M_BLK = 4096
N_BLK = M // M_BLK
N_CHUNK = M_BLK // 256


def _sims_body(tok_ref, keys_ref, sims_ref, m16_ref, m2_ref):
    i = pl.program_id(0)
    tok = tok_ref[...]
    keys = keys_ref[...]
    s = jax.lax.dot_general(tok, keys, (((1,), (1,)), ((), ())),
                            preferred_element_type=jnp.float32)
    sims_ref[...] = s
    m16 = s[:, 0:256]
    for j in range(1, N_CHUNK):
        m16 = jnp.maximum(m16, s[:, j * 256:(j + 1) * 256])
    m16_ref[0] = m16

    @pl.when(i == 0)
    def _():
        m2_ref[...] = m16

    @pl.when(i > 0)
    def _():
        m2_ref[...] = jnp.maximum(m2_ref[...], m16)


def _sims_stage(tokens, mem_keys):
    return pl.pallas_call(
        _sims_body,
        grid=(N_BLK,),
        in_specs=[
            pl.BlockSpec((T, D), lambda i: (0, 0)),
            pl.BlockSpec((M_BLK, D), lambda i: (i, 0)),
        ],
        out_specs=[
            pl.BlockSpec((T, M_BLK), lambda i: (0, i)),
            pl.BlockSpec((1, T, 256), lambda i: (i, 0, 0)),
            pl.BlockSpec((T, 256), lambda i: (0, 0)),
        ],
        out_shape=[
            jax.ShapeDtypeStruct((T, M), jnp.float32),
            jax.ShapeDtypeStruct((N_BLK, T, 256), jnp.float32),
            jax.ShapeDtypeStruct((T, 256), jnp.float32),
        ],
    )(tokens, mem_keys)


def _mlp_body(rec_ref, wn_ref, tok_ref, w1_ref, b1_ref, g_ref, bb_ref,
              w2_ref, b2_ref, out_ref):
    tb = tok_ref.shape[0]
    r = rec_ref[...].reshape(tb, TOPK, D)
    wn = wn_ref[...]
    ctx = jnp.sum(r * wn[:, :, None], axis=1)
    fused = tok_ref[...] + ctx
    h = jnp.dot(fused, w1_ref[...], preferred_element_type=jnp.float32) + b1_ref[...]
    mu = jnp.mean(h, axis=-1, keepdims=True)
    var = jnp.mean((h - mu) ** 2, axis=-1, keepdims=True)
    h = (h - mu) / jnp.sqrt(var + 1e-5) * g_ref[...] + bb_ref[...]
    h = jnp.maximum(h, 0.0)
    out_ref[...] = jnp.dot(h, w2_ref[...], preferred_element_type=jnp.float32) + b2_ref[...]


def _mlp_stage(recalled, wn, tokens, W1, b1, ln_g, ln_b, W2, b2):
    TB = 256
    nblk = T // TB
    full = lambda i: (0, 0)
    return pl.pallas_call(
        _mlp_body,
        grid=(nblk,),
        in_specs=[
            pl.BlockSpec((TB * TOPK, D), lambda i: (i, 0)),
            pl.BlockSpec((TB, TOPK), lambda i: (i, 0)),
            pl.BlockSpec((TB, D), lambda i: (i, 0)),
            pl.BlockSpec((D, D), full),
            pl.BlockSpec((1, D), full),
            pl.BlockSpec((1, D), full),
            pl.BlockSpec((1, D), full),
            pl.BlockSpec((D, D), full),
            pl.BlockSpec((1, D), full),
        ],
        out_specs=pl.BlockSpec((TB, D), lambda i: (i, 0)),
        out_shape=jax.ShapeDtypeStruct((T, D), jnp.float32),
    )(recalled, wn, tokens, W1, b1.reshape(1, D), ln_g.reshape(1, D),
      ln_b.reshape(1, D), W2, b2.reshape(1, D))


def kernel(input_tensor, mem_keys, mem_values, W1, b1, ln_g, ln_b, W2, b2):
    B, S, _ = input_tensor.shape
    tokens = input_tensor.reshape(T, D)

    sims, m16_all, m2 = _sims_stage(tokens, mem_keys)

    sl = jax.lax.top_k(m2, TOPK)[1]
    m16_r = m16_all.transpose(1, 0, 2)
    sl_b = jnp.broadcast_to(sl[:, None, :], (T, N_BLK, TOPK))
    cand1 = jnp.take_along_axis(m16_r, sl_b, axis=2)
    cand1 = cand1.transpose(0, 2, 1).reshape(T, TOPK * N_BLK)
    p = jax.lax.top_k(cand1, TOPK)[1]
    b_sel = p % N_BLK
    l_sel = jnp.take_along_axis(sl, p // N_BLK, axis=1)
    base = b_sel * M_BLK + l_sel
    eidx = (base[:, :, None] + jnp.arange(N_CHUNK)[None, None, :] * 256
            ).reshape(T, TOPK * N_CHUNK)
    cand2 = jnp.take_along_axis(sims, eidx, axis=1)
    w, pos = jax.lax.top_k(cand2, TOPK)
    final_idx = jnp.take_along_axis(eidx, pos, axis=1)
    wn = jax.nn.softmax(w, axis=-1)
    recalled = jnp.take(mem_values, final_idx.reshape(-1), axis=0)

    out = _mlp_stage(recalled, wn, tokens, W1, b1, ln_g, ln_b, W2, b2)
    return out.reshape(B, S, D)
